# baseline (speedup 1.0000x reference)
import functools

import jax
import jax.numpy as jnp
from jax import lax
from jax.experimental import pallas as pl
from jax.experimental.pallas import tpu as pltpu
from jax.experimental.pallas import tpu_sc as plsc

N = 10000
D = 128
E = 320000
W_AGG = 32
NW = 32
CPT_A = 82
CPT_B = 82
CPT_MAX = max(CPT_A, CPT_B)
NCHUNK = 16 * (CPT_A + CPT_B)
E_PAD = NCHUNK * 128
N_PAD = 10240
RPT = N_PAD // 16
BLK = 1024


def _sc_mesh():
    return plsc.VectorSubcoreMesh(core_axis_name="c", subcore_axis_name="s")


_SC_PARAMS = pltpu.CompilerParams(
    use_tc_tiling_on_sc=False, needs_layout_passes=False)


def _stage_chunks(hbm2, vmem, cid, sid):
    base_a = sid * CPT_A
    base_b = 16 * CPT_A + sid * CPT_B

    @pl.when(cid == 0)
    def _():
        pltpu.sync_copy(hbm2.at[pl.ds(base_a, CPT_A)], vmem.at[pl.ds(0, CPT_A)])

    @pl.when(cid == 1)
    def _():
        pltpu.sync_copy(hbm2.at[pl.ds(base_b, CPT_B)], vmem.at[pl.ds(0, CPT_B)])


def _deg_call(dst2):
    @functools.partial(
        pl.kernel,
        out_type=jax.ShapeDtypeStruct((NW, N_PAD), jnp.float32),
        mesh=_sc_mesh(),
        scratch_types=[
            pltpu.VMEM((CPT_MAX, 128), jnp.int32),
            pltpu.VMEM((N_PAD,), jnp.float32),
        ],
        compiler_params=_SC_PARAMS,
    )
    def deg_kernel(dst_hbm, degp_hbm, dstv, degacc):
        cid = lax.axis_index("c")
        sid = lax.axis_index("s")
        wid = sid * 2 + cid
        ncpt = jnp.where(cid == 0, CPT_A, CPT_B)
        _stage_chunks(dst_hbm, dstv, cid, sid)
        zero16 = jnp.zeros((16,), jnp.float32)

        @pl.loop(0, N_PAD // 16)
        def _zero(i):
            degacc[pl.ds(i * 16, 16)] = zero16

        one16 = jnp.ones((16,), jnp.float32)

        @pl.loop(0, ncpt)
        def _chunk(j):
            for t in range(8):
                idx = dstv[j, pl.ds(t * 16, 16)]
                plsc.addupdate_scatter(degacc, [idx], one16)

        pltpu.sync_copy(degacc, degp_hbm.at[wid])

    return deg_kernel(dst2)


def _agg_call(v, src3, dst3, dis=None):
    with_q = dis is not None
    outs = [jax.ShapeDtypeStruct((2, N_PAD, W_AGG), jnp.float32)]
    scratch = [
        pltpu.VMEM((CPT_MAX, 128), jnp.int32),
        pltpu.VMEM((CPT_MAX, 128), jnp.int32),
        [pltpu.VMEM((128, W_AGG), jnp.float32)] * 2,
        [pltpu.SemaphoreType.DMA] * 2,
        pltpu.VMEM_SHARED((N_PAD, W_AGG), jnp.float32),
    ]
    if with_q:
        outs.append(jax.ShapeDtypeStruct((NW, N_PAD), jnp.float32))
        scratch += [
            pltpu.VMEM((N_PAD,), jnp.float32),
            pltpu.VMEM((N_PAD,), jnp.float32),
        ]

    def body(refs):
        if with_q:
            (v_hbm, src_hbm, dst_hbm, dis_hbm, p_hbm, qp_hbm,
             srcv, dstv, rows, gsems, acc, disv, qacc) = refs
        else:
            (v_hbm, src_hbm, dst_hbm, p_hbm,
             srcv, dstv, rows, gsems, acc) = refs
        cid = lax.axis_index("c")
        sid = lax.axis_index("s")
        wid = sid * 2 + cid
        ncpt = jnp.where(cid == 0, CPT_A, CPT_B)
        _stage_chunks(src_hbm, srcv, cid, sid)
        _stage_chunks(dst_hbm, dstv, cid, sid)
        zero16 = jnp.zeros((16,), jnp.float32)

        @pl.loop(0, 128)
        def _zero_rows(i):
            rows[0][i, pl.ds(0, 16)] = zero16
            rows[0][i, pl.ds(16, 16)] = zero16

        for b in range(RPT // 128):
            pltpu.sync_copy(rows[0], acc.at[pl.ds(sid * RPT + b * 128, 128)])

        if with_q:
            pltpu.sync_copy(dis_hbm, disv)

            @pl.loop(0, N_PAD // 16)
            def _zero_q(i):
                qacc[pl.ds(i * 16, 16)] = zero16

        plsc.subcore_barrier()

        def gather(jj, b):
            pltpu.async_copy(v_hbm.at[srcv.at[jj]], rows[b], gsems[b])

        def gather_wait(jj, b):
            pltpu.make_async_copy(v_hbm.at[srcv.at[jj]], rows[b], gsems[b]).wait()

        gather(0, 0)
        gather(1, 1)

        @pl.loop(0, ncpt, step=2)
        def _chunk(j):
            for b in range(2):
                jj = j + b
                gather_wait(jj, b)
                pltpu.sync_copy(rows[b], acc.at[dstv.at[jj]], add=True)

                @pl.when(jj + 2 < ncpt)
                def _():
                    gather(jj + 2, b)

                if with_q:
                    for t in range(8):
                        si = srcv[jj, pl.ds(t * 16, 16)]
                        di = dstv[jj, pl.ds(t * 16, 16)]
                        dv = plsc.load_gather(disv, [si])
                        plsc.addupdate_scatter(qacc, [di], dv)

        plsc.subcore_barrier()
        pltpu.sync_copy(acc.at[pl.ds(sid * RPT, RPT)],
                        p_hbm.at[cid, pl.ds(sid * RPT, RPT)])
        if with_q:
            pltpu.sync_copy(qacc, qp_hbm.at[wid])

    def wrapped(*refs):
        body(refs)

    fn = functools.partial(
        pl.kernel,
        out_type=tuple(outs) if with_q else outs[0],
        mesh=_sc_mesh(),
        scratch_types=scratch,
        compiler_params=_SC_PARAMS,
    )(wrapped)
    if with_q:
        return fn(v, src3, dst3, dis)
    return fn(v, src3, dst3)


def _tc1(x_pad, W1, W2, degp):
    def body(x_ref, w1_ref, w2_ref, degp_ref, v1_ref, dis_ref):
        deg = jnp.sum(degp_ref[...], axis=0)
        dis = jnp.where(deg > 0, lax.rsqrt(deg), 0.0)
        w12 = jnp.dot(w1_ref[...], w2_ref[...], preferred_element_type=jnp.float32)
        c = jnp.dot(x_ref[...], w12, preferred_element_type=jnp.float32)
        v1_ref[...] = c * dis[:, None]
        dis_ref[...] = dis[None, :]

    return pl.pallas_call(
        body,
        grid=(N_PAD // BLK,),
        in_specs=[
            pl.BlockSpec((BLK, D), lambda i: (i, 0)),
            pl.BlockSpec((D, 64), lambda i: (0, 0)),
            pl.BlockSpec((64, W_AGG), lambda i: (0, 0)),
            pl.BlockSpec((NW, BLK), lambda i: (0, i)),
        ],
        out_specs=[
            pl.BlockSpec((BLK, W_AGG), lambda i: (i, 0)),
            pl.BlockSpec((1, BLK), lambda i: (0, i)),
        ],
        out_shape=[
            jax.ShapeDtypeStruct((N_PAD, W_AGG), jnp.float32),
            jax.ShapeDtypeStruct((1, N_PAD), jnp.float32),
        ],
    )(x_pad, W1, W2, degp)


def _tc2(p, qp, dis):
    def body(p_ref, qp_ref, dis_ref, v2_ref, q_ref):
        a = p_ref[0] + p_ref[1]
        d = dis_ref[0]
        v2_ref[...] = a * (d * d)[:, None]
        q_ref[...] = (d * jnp.sum(qp_ref[...], axis=0))[None, :]

    return pl.pallas_call(
        body,
        grid=(N_PAD // BLK,),
        in_specs=[
            pl.BlockSpec((2, BLK, W_AGG), lambda i: (0, i, 0)),
            pl.BlockSpec((NW, BLK), lambda i: (0, i)),
            pl.BlockSpec((1, BLK), lambda i: (0, i)),
        ],
        out_specs=[
            pl.BlockSpec((BLK, W_AGG), lambda i: (i, 0)),
            pl.BlockSpec((1, BLK), lambda i: (0, i)),
        ],
        out_shape=[
            jax.ShapeDtypeStruct((N_PAD, W_AGG), jnp.float32),
            jax.ShapeDtypeStruct((1, N_PAD), jnp.float32),
        ],
    )(p, qp, dis)


def _tc3(p, dis, q, b1r, W2, b2r):
    def body(p_ref, dis_ref, q_ref, b1_ref, w2_ref, b2_ref, v3_ref):
        d = dis_ref[0][:, None]
        qc = q_ref[0][:, None]
        bw = jnp.dot(b1_ref[...], w2_ref[...], preferred_element_type=jnp.float32)
        h2 = (p_ref[0] + p_ref[1]) * d + qc * bw + b2_ref[...]
        v3_ref[...] = jax.nn.sigmoid(h2) * d

    return pl.pallas_call(
        body,
        grid=(N_PAD // BLK,),
        in_specs=[
            pl.BlockSpec((2, BLK, W_AGG), lambda i: (0, i, 0)),
            pl.BlockSpec((1, BLK), lambda i: (0, i)),
            pl.BlockSpec((1, BLK), lambda i: (0, i)),
            pl.BlockSpec((1, 64), lambda i: (0, 0)),
            pl.BlockSpec((64, W_AGG), lambda i: (0, 0)),
            pl.BlockSpec((1, W_AGG), lambda i: (0, 0)),
        ],
        out_specs=pl.BlockSpec((BLK, W_AGG), lambda i: (i, 0)),
        out_shape=jax.ShapeDtypeStruct((N_PAD, W_AGG), jnp.float32),
    )(p, dis, q, b1r, W2, b2r)


def _tc4(p, dis):
    def body(p_ref, dis_ref, v4_ref):
        d = dis_ref[0]
        v4_ref[...] = (p_ref[0] + p_ref[1]) * (d * d)[:, None]

    return pl.pallas_call(
        body,
        grid=(N_PAD // BLK,),
        in_specs=[
            pl.BlockSpec((2, BLK, W_AGG), lambda i: (0, i, 0)),
            pl.BlockSpec((1, BLK), lambda i: (0, i)),
        ],
        out_specs=pl.BlockSpec((BLK, W_AGG), lambda i: (i, 0)),
        out_shape=jax.ShapeDtypeStruct((N_PAD, W_AGG), jnp.float32),
    )(p, dis)


def _tc5(p, dis, q, W3, W4, b3r, b4r):
    def body(p_ref, dis_ref, q_ref, w3_ref, w4_ref, b3_ref, b4_ref, out_ref):
        d = dis_ref[0][:, None]
        qc = q_ref[0][:, None]
        w34 = jnp.dot(w3_ref[...], w4_ref[...], preferred_element_type=jnp.float32)
        bw = jnp.dot(b3_ref[...], w4_ref[...], preferred_element_type=jnp.float32)
        t = (p_ref[0] + p_ref[1]) * d
        h4 = jnp.dot(t, w34, preferred_element_type=jnp.float32) + qc * bw + b4_ref[...]
        out_ref[...] = jnp.maximum(h4, 0.0)

    return pl.pallas_call(
        body,
        grid=(N_PAD // BLK,),
        in_specs=[
            pl.BlockSpec((2, BLK, W_AGG), lambda i: (0, i, 0)),
            pl.BlockSpec((1, BLK), lambda i: (0, i)),
            pl.BlockSpec((1, BLK), lambda i: (0, i)),
            pl.BlockSpec((W_AGG, 64), lambda i: (0, 0)),
            pl.BlockSpec((64, D), lambda i: (0, 0)),
            pl.BlockSpec((1, 64), lambda i: (0, 0)),
            pl.BlockSpec((1, D), lambda i: (0, 0)),
        ],
        out_specs=pl.BlockSpec((BLK, D), lambda i: (i, 0)),
        out_shape=jax.ShapeDtypeStruct((N_PAD, D), jnp.float32),
    )(p, dis, q, W3, W4, b3r, b4r)


def kernel(x, edge_index, edge_weights, W1, b1, W2, b2, W3, b3, W4, b4):
    del edge_weights
    loop = jnp.arange(N, dtype=jnp.int32)
    pad = jnp.full((E_PAD - E - N,), N, dtype=jnp.int32)
    src3 = jnp.concatenate([edge_index[0], loop, pad]).reshape(NCHUNK, 128)
    dst3 = jnp.concatenate([edge_index[1], loop, pad]).reshape(NCHUNK, 128)
    x_pad = jnp.pad(x, ((0, N_PAD - N), (0, 0)))

    degp = _deg_call(dst3)
    v1, dis = _tc1(x_pad, W1, W2, degp)
    p, qp = _agg_call(v1, src3, dst3, dis=dis.reshape(N_PAD))
    v2, q = _tc2(p, qp, dis)
    p = _agg_call(v2, src3, dst3)
    v3 = _tc3(p, dis, q, b1.reshape(1, 64), W2, b2.reshape(1, W_AGG))
    p = _agg_call(v3, src3, dst3)
    v4 = _tc4(p, dis)
    p = _agg_call(v4, src3, dst3)
    out = _tc5(p, dis, q, W3, W4, b3.reshape(1, 64), b4.reshape(1, D))
    return out[:N]


# speedup vs baseline: 1.3506x; 1.3506x over previous
import functools

import jax
import jax.numpy as jnp
from jax import lax
from jax.experimental import pallas as pl
from jax.experimental.pallas import tpu as pltpu
from jax.experimental.pallas import tpu_sc as plsc

N = 10000
D = 128
E = 320000
W_AGG = 32
NW = 32
CPT_A = 78
CPT_B = 84
CPT_MAX = max(CPT_A, CPT_B)
NCHUNK = 16 * (CPT_A + CPT_B)
E_PAD = NCHUNK * 128
N_PAD = 10240
RPT = N_PAD // 16
BLK = 1024


def _sc_mesh():
    return plsc.VectorSubcoreMesh(core_axis_name="c", subcore_axis_name="s")


_SC_PARAMS = pltpu.CompilerParams(
    use_tc_tiling_on_sc=False, needs_layout_passes=False)


def _stage_chunks(hbm2, vmem, cid, sid):
    base_a = sid * CPT_A
    base_b = 16 * CPT_A + sid * CPT_B

    @pl.when(cid == 0)
    def _():
        pltpu.sync_copy(hbm2.at[pl.ds(base_a, CPT_A)], vmem.at[pl.ds(0, CPT_A)])

    @pl.when(cid == 1)
    def _():
        pltpu.sync_copy(hbm2.at[pl.ds(base_b, CPT_B)], vmem.at[pl.ds(0, CPT_B)])


def _deg_call(dst2):
    @functools.partial(
        pl.kernel,
        out_type=jax.ShapeDtypeStruct((NW, N_PAD), jnp.float32),
        mesh=_sc_mesh(),
        scratch_types=[
            pltpu.VMEM((CPT_MAX, 128), jnp.int32),
            pltpu.VMEM((N_PAD,), jnp.float32),
        ],
        compiler_params=_SC_PARAMS,
    )
    def deg_kernel(dst_hbm, degp_hbm, dstv, degacc):
        cid = lax.axis_index("c")
        sid = lax.axis_index("s")
        wid = sid * 2 + cid
        ncpt = jnp.where(cid == 0, CPT_A, CPT_B)
        _stage_chunks(dst_hbm, dstv, cid, sid)
        zero16 = jnp.zeros((16,), jnp.float32)

        @pl.loop(0, N_PAD // 16)
        def _zero(i):
            degacc[pl.ds(i * 16, 16)] = zero16

        one16 = jnp.ones((16,), jnp.float32)

        @pl.loop(0, ncpt)
        def _chunk(j):
            for t in range(8):
                idx = dstv[j, pl.ds(t * 16, 16)]
                plsc.addupdate_scatter(degacc, [idx], one16)

        pltpu.sync_copy(degacc, degp_hbm.at[wid])

    return deg_kernel(dst2)


def _agg_call(v, src3, dst3, dis=None):
    with_q = dis is not None
    outs = [jax.ShapeDtypeStruct((2, N_PAD, W_AGG), jnp.float32)]
    scratch = [
        pltpu.VMEM((CPT_MAX, 128), jnp.int32),
        pltpu.VMEM((CPT_MAX, 128), jnp.int32),
        [pltpu.VMEM((128, W_AGG), jnp.float32)] * 2,
        [pltpu.SemaphoreType.DMA] * 2,
        pltpu.VMEM_SHARED((N_PAD, W_AGG), jnp.float32),
    ]
    if with_q:
        outs.append(jax.ShapeDtypeStruct((NW, N_PAD), jnp.float32))
        scratch += [
            pltpu.VMEM((N_PAD,), jnp.float32),
            pltpu.VMEM((N_PAD,), jnp.float32),
        ]

    def body(refs):
        if with_q:
            (v_hbm, src_hbm, dst_hbm, dis_hbm, p_hbm, qp_hbm,
             srcv, dstv, rows, gsems, acc, disv, qacc) = refs
        else:
            (v_hbm, src_hbm, dst_hbm, p_hbm,
             srcv, dstv, rows, gsems, acc) = refs
        cid = lax.axis_index("c")
        sid = lax.axis_index("s")
        wid = sid * 2 + cid
        ncpt = jnp.where(cid == 0, CPT_A, CPT_B)
        _stage_chunks(src_hbm, srcv, cid, sid)
        _stage_chunks(dst_hbm, dstv, cid, sid)
        zero16 = jnp.zeros((16,), jnp.float32)

        @pl.loop(0, 128)
        def _zero_rows(i):
            rows[0][i, pl.ds(0, 16)] = zero16
            rows[0][i, pl.ds(16, 16)] = zero16

        for b in range(RPT // 128):
            pltpu.sync_copy(rows[0], acc.at[pl.ds(sid * RPT + b * 128, 128)])

        if with_q:
            pltpu.sync_copy(dis_hbm, disv)

            @pl.loop(0, N_PAD // 16)
            def _zero_q(i):
                qacc[pl.ds(i * 16, 16)] = zero16

        plsc.subcore_barrier()

        def gather(jj, b):
            pltpu.async_copy(v_hbm.at[srcv.at[jj]], rows[b], gsems[b])

        def gather_wait(jj, b):
            pltpu.make_async_copy(v_hbm.at[srcv.at[jj]], rows[b], gsems[b]).wait()

        gather(0, 0)
        gather(1, 1)

        @pl.loop(0, ncpt, step=2)
        def _chunk(j):
            for b in range(2):
                jj = j + b
                gather_wait(jj, b)
                pltpu.sync_copy(rows[b], acc.at[dstv.at[jj]], add=True)

                @pl.when(jj + 2 < ncpt)
                def _():
                    gather(jj + 2, b)

                if with_q:
                    for t in range(8):
                        si = srcv[jj, pl.ds(t * 16, 16)]
                        di = dstv[jj, pl.ds(t * 16, 16)]
                        dv = plsc.load_gather(disv, [si])
                        plsc.addupdate_scatter(qacc, [di], dv)

        plsc.subcore_barrier()
        pltpu.sync_copy(acc.at[pl.ds(sid * RPT, RPT)],
                        p_hbm.at[cid, pl.ds(sid * RPT, RPT)])
        if with_q:
            pltpu.sync_copy(qacc, qp_hbm.at[wid])

    def wrapped(*refs):
        body(refs)

    fn = functools.partial(
        pl.kernel,
        out_type=tuple(outs) if with_q else outs[0],
        mesh=_sc_mesh(),
        scratch_types=scratch,
        compiler_params=_SC_PARAMS,
    )(wrapped)
    if with_q:
        return fn(v, src3, dst3, dis)
    return fn(v, src3, dst3)


def _tc1(x_pad, W1, W2, degp):
    def body(x_ref, w1_ref, w2_ref, degp_ref, v1_ref, dis_ref):
        deg = jnp.sum(degp_ref[...], axis=0)
        dis = jnp.where(deg > 0, lax.rsqrt(deg), 0.0)
        w12 = jnp.dot(w1_ref[...], w2_ref[...], preferred_element_type=jnp.float32)
        c = jnp.dot(x_ref[...], w12, preferred_element_type=jnp.float32)
        v1_ref[...] = c * dis[:, None]
        dis_ref[...] = dis[None, :]

    return pl.pallas_call(
        body,
        grid=(N_PAD // BLK,),
        in_specs=[
            pl.BlockSpec((BLK, D), lambda i: (i, 0)),
            pl.BlockSpec((D, 64), lambda i: (0, 0)),
            pl.BlockSpec((64, W_AGG), lambda i: (0, 0)),
            pl.BlockSpec((NW, BLK), lambda i: (0, i)),
        ],
        out_specs=[
            pl.BlockSpec((BLK, W_AGG), lambda i: (i, 0)),
            pl.BlockSpec((1, BLK), lambda i: (0, i)),
        ],
        out_shape=[
            jax.ShapeDtypeStruct((N_PAD, W_AGG), jnp.float32),
            jax.ShapeDtypeStruct((1, N_PAD), jnp.float32),
        ],
    )(x_pad, W1, W2, degp)


def _tc2(p, qp, dis):
    def body(p_ref, qp_ref, dis_ref, v2_ref, q_ref):
        a = p_ref[0] + p_ref[1]
        d = dis_ref[0]
        v2_ref[...] = a * (d * d)[:, None]
        q_ref[...] = (d * jnp.sum(qp_ref[...], axis=0))[None, :]

    return pl.pallas_call(
        body,
        grid=(N_PAD // BLK,),
        in_specs=[
            pl.BlockSpec((2, BLK, W_AGG), lambda i: (0, i, 0)),
            pl.BlockSpec((NW, BLK), lambda i: (0, i)),
            pl.BlockSpec((1, BLK), lambda i: (0, i)),
        ],
        out_specs=[
            pl.BlockSpec((BLK, W_AGG), lambda i: (i, 0)),
            pl.BlockSpec((1, BLK), lambda i: (0, i)),
        ],
        out_shape=[
            jax.ShapeDtypeStruct((N_PAD, W_AGG), jnp.float32),
            jax.ShapeDtypeStruct((1, N_PAD), jnp.float32),
        ],
    )(p, qp, dis)


def _tc3(p, dis, q, b1r, W2, b2r):
    def body(p_ref, dis_ref, q_ref, b1_ref, w2_ref, b2_ref, v3_ref):
        d = dis_ref[0][:, None]
        qc = q_ref[0][:, None]
        bw = jnp.dot(b1_ref[...], w2_ref[...], preferred_element_type=jnp.float32)
        h2 = (p_ref[0] + p_ref[1]) * d + qc * bw + b2_ref[...]
        v3_ref[...] = jax.nn.sigmoid(h2) * d

    return pl.pallas_call(
        body,
        grid=(N_PAD // BLK,),
        in_specs=[
            pl.BlockSpec((2, BLK, W_AGG), lambda i: (0, i, 0)),
            pl.BlockSpec((1, BLK), lambda i: (0, i)),
            pl.BlockSpec((1, BLK), lambda i: (0, i)),
            pl.BlockSpec((1, 64), lambda i: (0, 0)),
            pl.BlockSpec((64, W_AGG), lambda i: (0, 0)),
            pl.BlockSpec((1, W_AGG), lambda i: (0, 0)),
        ],
        out_specs=pl.BlockSpec((BLK, W_AGG), lambda i: (i, 0)),
        out_shape=jax.ShapeDtypeStruct((N_PAD, W_AGG), jnp.float32),
    )(p, dis, q, b1r, W2, b2r)


def _tc4(p, dis):
    def body(p_ref, dis_ref, v4_ref):
        d = dis_ref[0]
        v4_ref[...] = (p_ref[0] + p_ref[1]) * (d * d)[:, None]

    return pl.pallas_call(
        body,
        grid=(N_PAD // BLK,),
        in_specs=[
            pl.BlockSpec((2, BLK, W_AGG), lambda i: (0, i, 0)),
            pl.BlockSpec((1, BLK), lambda i: (0, i)),
        ],
        out_specs=pl.BlockSpec((BLK, W_AGG), lambda i: (i, 0)),
        out_shape=jax.ShapeDtypeStruct((N_PAD, W_AGG), jnp.float32),
    )(p, dis)


def _tc5(p, dis, q, W3, W4, b3r, b4r):
    def body(p_ref, dis_ref, q_ref, w3_ref, w4_ref, b3_ref, b4_ref, out_ref):
        d = dis_ref[0][:, None]
        qc = q_ref[0][:, None]
        w34 = jnp.dot(w3_ref[...], w4_ref[...], preferred_element_type=jnp.float32)
        bw = jnp.dot(b3_ref[...], w4_ref[...], preferred_element_type=jnp.float32)
        t = (p_ref[0] + p_ref[1]) * d
        h4 = jnp.dot(t, w34, preferred_element_type=jnp.float32) + qc * bw + b4_ref[...]
        out_ref[...] = jnp.maximum(h4, 0.0)

    return pl.pallas_call(
        body,
        grid=(N_PAD // BLK,),
        in_specs=[
            pl.BlockSpec((2, BLK, W_AGG), lambda i: (0, i, 0)),
            pl.BlockSpec((1, BLK), lambda i: (0, i)),
            pl.BlockSpec((1, BLK), lambda i: (0, i)),
            pl.BlockSpec((W_AGG, 64), lambda i: (0, 0)),
            pl.BlockSpec((64, D), lambda i: (0, 0)),
            pl.BlockSpec((1, 64), lambda i: (0, 0)),
            pl.BlockSpec((1, D), lambda i: (0, 0)),
        ],
        out_specs=pl.BlockSpec((BLK, D), lambda i: (i, 0)),
        out_shape=jax.ShapeDtypeStruct((N_PAD, D), jnp.float32),
    )(p, dis, q, W3, W4, b3r, b4r)


def kernel(x, edge_index, edge_weights, W1, b1, W2, b2, W3, b3, W4, b4):
    del edge_weights
    loop = jnp.arange(N, dtype=jnp.int32)
    pad = jnp.full((E_PAD - E - N,), N, dtype=jnp.int32)
    src3 = jnp.concatenate([edge_index[0], loop, pad]).reshape(NCHUNK, 128)
    dst3 = jnp.concatenate([edge_index[1], loop, pad]).reshape(NCHUNK, 128)
    x_pad = jnp.pad(x, ((0, N_PAD - N), (0, 0)))

    degp = _deg_call(dst3)
    v1, dis = _tc1(x_pad, W1, W2, degp)
    p, qp = _agg_call(v1, src3, dst3, dis=dis.reshape(N_PAD))
    v2, q = _tc2(p, qp, dis)
    p = _agg_call(v2, src3, dst3)
    v3 = _tc3(p, dis, q, b1.reshape(1, 64), W2, b2.reshape(1, W_AGG))
    p = _agg_call(v3, src3, dst3)
    v4 = _tc4(p, dis)
    p = _agg_call(v4, src3, dst3)
    out = _tc5(p, dis, q, W3, W4, b3.reshape(1, 64), b4.reshape(1, D))
    return out[:N]


# speedup vs baseline: 1.3662x; 1.0116x over previous
import functools

import jax
import jax.numpy as jnp
from jax import lax
from jax.experimental import pallas as pl
from jax.experimental.pallas import tpu as pltpu
from jax.experimental.pallas import tpu_sc as plsc

N = 10000
D = 128
E = 320000
W_AGG = 32
NW = 32
CPT_A = 80
CPT_B = 82
CPT_MAX = max(CPT_A, CPT_B)
NCHUNK = 16 * (CPT_A + CPT_B)
E_PAD = NCHUNK * 128
N_PAD = 10240
RPT = N_PAD // 16
BLK = 1024


def _sc_mesh():
    return plsc.VectorSubcoreMesh(core_axis_name="c", subcore_axis_name="s")


_SC_PARAMS = pltpu.CompilerParams(
    use_tc_tiling_on_sc=False, needs_layout_passes=False)


def _stage_chunks(hbm2, vmem, cid, sid):
    base_a = sid * CPT_A
    base_b = 16 * CPT_A + sid * CPT_B

    @pl.when(cid == 0)
    def _():
        pltpu.sync_copy(hbm2.at[pl.ds(base_a, CPT_A)], vmem.at[pl.ds(0, CPT_A)])

    @pl.when(cid == 1)
    def _():
        pltpu.sync_copy(hbm2.at[pl.ds(base_b, CPT_B)], vmem.at[pl.ds(0, CPT_B)])


def _deg_call(dst2):
    @functools.partial(
        pl.kernel,
        out_type=jax.ShapeDtypeStruct((NW, N_PAD), jnp.float32),
        mesh=_sc_mesh(),
        scratch_types=[
            pltpu.VMEM((CPT_MAX, 128), jnp.int32),
            pltpu.VMEM((N_PAD,), jnp.float32),
        ],
        compiler_params=_SC_PARAMS,
    )
    def deg_kernel(dst_hbm, degp_hbm, dstv, degacc):
        cid = lax.axis_index("c")
        sid = lax.axis_index("s")
        wid = sid * 2 + cid
        ncpt = jnp.where(cid == 0, CPT_A, CPT_B)
        _stage_chunks(dst_hbm, dstv, cid, sid)
        zero16 = jnp.zeros((16,), jnp.float32)

        @pl.loop(0, N_PAD // 16)
        def _zero(i):
            degacc[pl.ds(i * 16, 16)] = zero16

        one16 = jnp.ones((16,), jnp.float32)

        @pl.loop(0, ncpt)
        def _chunk(j):
            for t in range(8):
                idx = dstv[j, pl.ds(t * 16, 16)]
                plsc.addupdate_scatter(degacc, [idx], one16)

        pltpu.sync_copy(degacc, degp_hbm.at[wid])

    return deg_kernel(dst2)


def _agg_call(v, src3, dst3, dis=None):
    with_q = dis is not None
    outs = [jax.ShapeDtypeStruct((2, N_PAD, W_AGG), jnp.float32)]
    scratch = [
        pltpu.VMEM((CPT_MAX, 128), jnp.int32),
        pltpu.VMEM((CPT_MAX, 128), jnp.int32),
        [pltpu.VMEM((128, W_AGG), jnp.float32)] * 2,
        [pltpu.SemaphoreType.DMA] * 2,
        pltpu.VMEM_SHARED((N_PAD, W_AGG), jnp.float32),
    ]
    if with_q:
        outs.append(jax.ShapeDtypeStruct((NW, N_PAD), jnp.float32))
        scratch += [
            pltpu.VMEM((N_PAD,), jnp.float32),
            pltpu.VMEM((N_PAD,), jnp.float32),
        ]

    def body(refs):
        if with_q:
            (v_hbm, src_hbm, dst_hbm, dis_hbm, p_hbm, qp_hbm,
             srcv, dstv, rows, gsems, acc, disv, qacc) = refs
        else:
            (v_hbm, src_hbm, dst_hbm, p_hbm,
             srcv, dstv, rows, gsems, acc) = refs
        cid = lax.axis_index("c")
        sid = lax.axis_index("s")
        wid = sid * 2 + cid
        ncpt = jnp.where(cid == 0, CPT_A, CPT_B)
        _stage_chunks(src_hbm, srcv, cid, sid)
        _stage_chunks(dst_hbm, dstv, cid, sid)
        zero16 = jnp.zeros((16,), jnp.float32)

        @pl.loop(0, 128)
        def _zero_rows(i):
            rows[0][i, pl.ds(0, 16)] = zero16
            rows[0][i, pl.ds(16, 16)] = zero16

        for b in range(RPT // 128):
            pltpu.sync_copy(rows[0], acc.at[pl.ds(sid * RPT + b * 128, 128)])

        if with_q:
            pltpu.sync_copy(dis_hbm, disv)

            @pl.loop(0, N_PAD // 16)
            def _zero_q(i):
                qacc[pl.ds(i * 16, 16)] = zero16

        plsc.subcore_barrier()

        def gather(jj, b):
            pltpu.async_copy(v_hbm.at[srcv.at[jj]], rows[b], gsems[b])

        def gather_wait(jj, b):
            pltpu.make_async_copy(v_hbm.at[srcv.at[jj]], rows[b], gsems[b]).wait()

        gather(0, 0)
        gather(1, 1)

        @pl.loop(0, ncpt, step=2)
        def _chunk(j):
            for b in range(2):
                jj = j + b
                gather_wait(jj, b)
                pltpu.sync_copy(rows[b], acc.at[dstv.at[jj]], add=True)

                @pl.when(jj + 2 < ncpt)
                def _():
                    gather(jj + 2, b)

                if with_q:
                    for t in range(8):
                        si = srcv[jj, pl.ds(t * 16, 16)]
                        di = dstv[jj, pl.ds(t * 16, 16)]
                        dv = plsc.load_gather(disv, [si])
                        plsc.addupdate_scatter(qacc, [di], dv)

        plsc.subcore_barrier()
        pltpu.sync_copy(acc.at[pl.ds(sid * RPT, RPT)],
                        p_hbm.at[cid, pl.ds(sid * RPT, RPT)])
        if with_q:
            pltpu.sync_copy(qacc, qp_hbm.at[wid])

    def wrapped(*refs):
        body(refs)

    fn = functools.partial(
        pl.kernel,
        out_type=tuple(outs) if with_q else outs[0],
        mesh=_sc_mesh(),
        scratch_types=scratch,
        compiler_params=_SC_PARAMS,
    )(wrapped)
    if with_q:
        return fn(v, src3, dst3, dis)
    return fn(v, src3, dst3)


def _tc1(x_pad, W1, W2, degp):
    def body(x_ref, w1_ref, w2_ref, degp_ref, v1_ref, dis_ref):
        deg = jnp.sum(degp_ref[...], axis=0)
        dis = jnp.where(deg > 0, lax.rsqrt(deg), 0.0)
        w12 = jnp.dot(w1_ref[...], w2_ref[...], preferred_element_type=jnp.float32)
        c = jnp.dot(x_ref[...], w12, preferred_element_type=jnp.float32)
        v1_ref[...] = c * dis[:, None]
        dis_ref[...] = dis[None, :]

    return pl.pallas_call(
        body,
        grid=(N_PAD // BLK,),
        in_specs=[
            pl.BlockSpec((BLK, D), lambda i: (i, 0)),
            pl.BlockSpec((D, 64), lambda i: (0, 0)),
            pl.BlockSpec((64, W_AGG), lambda i: (0, 0)),
            pl.BlockSpec((NW, BLK), lambda i: (0, i)),
        ],
        out_specs=[
            pl.BlockSpec((BLK, W_AGG), lambda i: (i, 0)),
            pl.BlockSpec((1, BLK), lambda i: (0, i)),
        ],
        out_shape=[
            jax.ShapeDtypeStruct((N_PAD, W_AGG), jnp.float32),
            jax.ShapeDtypeStruct((1, N_PAD), jnp.float32),
        ],
    )(x_pad, W1, W2, degp)


def _tc2(p, qp, dis):
    def body(p_ref, qp_ref, dis_ref, v2_ref, q_ref):
        a = p_ref[0] + p_ref[1]
        d = dis_ref[0]
        v2_ref[...] = a * (d * d)[:, None]
        q_ref[...] = (d * jnp.sum(qp_ref[...], axis=0))[None, :]

    return pl.pallas_call(
        body,
        grid=(N_PAD // BLK,),
        in_specs=[
            pl.BlockSpec((2, BLK, W_AGG), lambda i: (0, i, 0)),
            pl.BlockSpec((NW, BLK), lambda i: (0, i)),
            pl.BlockSpec((1, BLK), lambda i: (0, i)),
        ],
        out_specs=[
            pl.BlockSpec((BLK, W_AGG), lambda i: (i, 0)),
            pl.BlockSpec((1, BLK), lambda i: (0, i)),
        ],
        out_shape=[
            jax.ShapeDtypeStruct((N_PAD, W_AGG), jnp.float32),
            jax.ShapeDtypeStruct((1, N_PAD), jnp.float32),
        ],
    )(p, qp, dis)


def _tc3(p, dis, q, b1r, W2, b2r):
    def body(p_ref, dis_ref, q_ref, b1_ref, w2_ref, b2_ref, v3_ref):
        d = dis_ref[0][:, None]
        qc = q_ref[0][:, None]
        bw = jnp.dot(b1_ref[...], w2_ref[...], preferred_element_type=jnp.float32)
        h2 = (p_ref[0] + p_ref[1]) * d + qc * bw + b2_ref[...]
        v3_ref[...] = jax.nn.sigmoid(h2) * d

    return pl.pallas_call(
        body,
        grid=(N_PAD // BLK,),
        in_specs=[
            pl.BlockSpec((2, BLK, W_AGG), lambda i: (0, i, 0)),
            pl.BlockSpec((1, BLK), lambda i: (0, i)),
            pl.BlockSpec((1, BLK), lambda i: (0, i)),
            pl.BlockSpec((1, 64), lambda i: (0, 0)),
            pl.BlockSpec((64, W_AGG), lambda i: (0, 0)),
            pl.BlockSpec((1, W_AGG), lambda i: (0, 0)),
        ],
        out_specs=pl.BlockSpec((BLK, W_AGG), lambda i: (i, 0)),
        out_shape=jax.ShapeDtypeStruct((N_PAD, W_AGG), jnp.float32),
    )(p, dis, q, b1r, W2, b2r)


def _tc4(p, dis):
    def body(p_ref, dis_ref, v4_ref):
        d = dis_ref[0]
        v4_ref[...] = (p_ref[0] + p_ref[1]) * (d * d)[:, None]

    return pl.pallas_call(
        body,
        grid=(N_PAD // BLK,),
        in_specs=[
            pl.BlockSpec((2, BLK, W_AGG), lambda i: (0, i, 0)),
            pl.BlockSpec((1, BLK), lambda i: (0, i)),
        ],
        out_specs=pl.BlockSpec((BLK, W_AGG), lambda i: (i, 0)),
        out_shape=jax.ShapeDtypeStruct((N_PAD, W_AGG), jnp.float32),
    )(p, dis)


def _tc5(p, dis, q, W3, W4, b3r, b4r):
    def body(p_ref, dis_ref, q_ref, w3_ref, w4_ref, b3_ref, b4_ref, out_ref):
        d = dis_ref[0][:, None]
        qc = q_ref[0][:, None]
        w34 = jnp.dot(w3_ref[...], w4_ref[...], preferred_element_type=jnp.float32)
        bw = jnp.dot(b3_ref[...], w4_ref[...], preferred_element_type=jnp.float32)
        t = (p_ref[0] + p_ref[1]) * d
        h4 = jnp.dot(t, w34, preferred_element_type=jnp.float32) + qc * bw + b4_ref[...]
        out_ref[...] = jnp.maximum(h4, 0.0)

    return pl.pallas_call(
        body,
        grid=(N_PAD // BLK,),
        in_specs=[
            pl.BlockSpec((2, BLK, W_AGG), lambda i: (0, i, 0)),
            pl.BlockSpec((1, BLK), lambda i: (0, i)),
            pl.BlockSpec((1, BLK), lambda i: (0, i)),
            pl.BlockSpec((W_AGG, 64), lambda i: (0, 0)),
            pl.BlockSpec((64, D), lambda i: (0, 0)),
            pl.BlockSpec((1, 64), lambda i: (0, 0)),
            pl.BlockSpec((1, D), lambda i: (0, 0)),
        ],
        out_specs=pl.BlockSpec((BLK, D), lambda i: (i, 0)),
        out_shape=jax.ShapeDtypeStruct((N_PAD, D), jnp.float32),
    )(p, dis, q, W3, W4, b3r, b4r)


def kernel(x, edge_index, edge_weights, W1, b1, W2, b2, W3, b3, W4, b4):
    del edge_weights
    loop = jnp.arange(N, dtype=jnp.int32)
    pad = jnp.full((E_PAD - E - N,), N, dtype=jnp.int32)
    src3 = jnp.concatenate([edge_index[0], loop, pad]).reshape(NCHUNK, 128)
    dst3 = jnp.concatenate([edge_index[1], loop, pad]).reshape(NCHUNK, 128)
    x_pad = jnp.pad(x, ((0, N_PAD - N), (0, 0)))

    degp = _deg_call(dst3)
    v1, dis = _tc1(x_pad, W1, W2, degp)
    p, qp = _agg_call(v1, src3, dst3, dis=dis.reshape(N_PAD))
    v2, q = _tc2(p, qp, dis)
    p = _agg_call(v2, src3, dst3)
    v3 = _tc3(p, dis, q, b1.reshape(1, 64), W2, b2.reshape(1, W_AGG))
    p = _agg_call(v3, src3, dst3)
    v4 = _tc4(p, dis)
    p = _agg_call(v4, src3, dst3)
    out = _tc5(p, dis, q, W3, W4, b3.reshape(1, 64), b4.reshape(1, D))
    return out[:N]


# speedup vs baseline: 1.3797x; 1.0099x over previous
import functools

import jax
import jax.numpy as jnp
from jax import lax
from jax.experimental import pallas as pl
from jax.experimental.pallas import tpu as pltpu
from jax.experimental.pallas import tpu_sc as plsc

N = 10000
D = 128
E = 320000
W_AGG = 32
NW = 32
CPT_A = 82
CPT_B = 80
CPT_MAX = max(CPT_A, CPT_B)
NCHUNK = 16 * (CPT_A + CPT_B)
E_PAD = NCHUNK * 128
N_PAD = 10240
RPT = N_PAD // 16
BLK = 1024


def _sc_mesh():
    return plsc.VectorSubcoreMesh(core_axis_name="c", subcore_axis_name="s")


_SC_PARAMS = pltpu.CompilerParams(
    use_tc_tiling_on_sc=False, needs_layout_passes=False)


def _stage_chunks(hbm2, vmem, cid, sid):
    base_a = sid * CPT_A
    base_b = 16 * CPT_A + sid * CPT_B

    @pl.when(cid == 0)
    def _():
        pltpu.sync_copy(hbm2.at[pl.ds(base_a, CPT_A)], vmem.at[pl.ds(0, CPT_A)])

    @pl.when(cid == 1)
    def _():
        pltpu.sync_copy(hbm2.at[pl.ds(base_b, CPT_B)], vmem.at[pl.ds(0, CPT_B)])


def _deg_call(dst2):
    @functools.partial(
        pl.kernel,
        out_type=jax.ShapeDtypeStruct((NW, N_PAD), jnp.float32),
        mesh=_sc_mesh(),
        scratch_types=[
            pltpu.VMEM((CPT_MAX, 128), jnp.int32),
            pltpu.VMEM((N_PAD,), jnp.float32),
        ],
        compiler_params=_SC_PARAMS,
    )
    def deg_kernel(dst_hbm, degp_hbm, dstv, degacc):
        cid = lax.axis_index("c")
        sid = lax.axis_index("s")
        wid = sid * 2 + cid
        ncpt = jnp.where(cid == 0, CPT_A, CPT_B)
        _stage_chunks(dst_hbm, dstv, cid, sid)
        zero16 = jnp.zeros((16,), jnp.float32)

        @pl.loop(0, N_PAD // 16)
        def _zero(i):
            degacc[pl.ds(i * 16, 16)] = zero16

        one16 = jnp.ones((16,), jnp.float32)

        @pl.loop(0, ncpt)
        def _chunk(j):
            for t in range(8):
                idx = dstv[j, pl.ds(t * 16, 16)]
                plsc.addupdate_scatter(degacc, [idx], one16)

        pltpu.sync_copy(degacc, degp_hbm.at[wid])

    return deg_kernel(dst2)


def _agg_call(v, src3, dst3, dis=None):
    with_q = dis is not None
    outs = [jax.ShapeDtypeStruct((2, N_PAD, W_AGG), jnp.float32)]
    scratch = [
        pltpu.VMEM((CPT_MAX, 128), jnp.int32),
        pltpu.VMEM((CPT_MAX, 128), jnp.int32),
        [pltpu.VMEM((128, W_AGG), jnp.float32)] * 2,
        [pltpu.SemaphoreType.DMA] * 2,
        pltpu.VMEM_SHARED((N_PAD, W_AGG), jnp.float32),
    ]
    if with_q:
        outs.append(jax.ShapeDtypeStruct((NW, N_PAD), jnp.float32))
        scratch += [
            pltpu.VMEM((N_PAD,), jnp.float32),
            pltpu.VMEM((N_PAD,), jnp.float32),
        ]

    def body(refs):
        if with_q:
            (v_hbm, src_hbm, dst_hbm, dis_hbm, p_hbm, qp_hbm,
             srcv, dstv, rows, gsems, acc, disv, qacc) = refs
        else:
            (v_hbm, src_hbm, dst_hbm, p_hbm,
             srcv, dstv, rows, gsems, acc) = refs
        cid = lax.axis_index("c")
        sid = lax.axis_index("s")
        wid = sid * 2 + cid
        ncpt = jnp.where(cid == 0, CPT_A, CPT_B)
        _stage_chunks(src_hbm, srcv, cid, sid)
        _stage_chunks(dst_hbm, dstv, cid, sid)
        zero16 = jnp.zeros((16,), jnp.float32)

        @pl.loop(0, 128)
        def _zero_rows(i):
            rows[0][i, pl.ds(0, 16)] = zero16
            rows[0][i, pl.ds(16, 16)] = zero16

        for b in range(RPT // 128):
            pltpu.sync_copy(rows[0], acc.at[pl.ds(sid * RPT + b * 128, 128)])

        if with_q:
            pltpu.sync_copy(dis_hbm, disv)

            @pl.loop(0, N_PAD // 16)
            def _zero_q(i):
                qacc[pl.ds(i * 16, 16)] = zero16

        plsc.subcore_barrier()

        def gather(jj, b):
            pltpu.async_copy(v_hbm.at[srcv.at[jj]], rows[b], gsems[b])

        def gather_wait(jj, b):
            pltpu.make_async_copy(v_hbm.at[srcv.at[jj]], rows[b], gsems[b]).wait()

        gather(0, 0)
        gather(1, 1)

        @pl.loop(0, ncpt, step=2)
        def _chunk(j):
            for b in range(2):
                jj = j + b
                gather_wait(jj, b)
                pltpu.sync_copy(rows[b], acc.at[dstv.at[jj]], add=True)

                @pl.when(jj + 2 < ncpt)
                def _():
                    gather(jj + 2, b)

                if with_q:
                    for t in range(8):
                        si = srcv[jj, pl.ds(t * 16, 16)]
                        di = dstv[jj, pl.ds(t * 16, 16)]
                        dv = plsc.load_gather(disv, [si])
                        plsc.addupdate_scatter(qacc, [di], dv)

        plsc.subcore_barrier()
        pltpu.sync_copy(acc.at[pl.ds(sid * RPT, RPT)],
                        p_hbm.at[cid, pl.ds(sid * RPT, RPT)])
        if with_q:
            pltpu.sync_copy(qacc, qp_hbm.at[wid])

    def wrapped(*refs):
        body(refs)

    fn = functools.partial(
        pl.kernel,
        out_type=tuple(outs) if with_q else outs[0],
        mesh=_sc_mesh(),
        scratch_types=scratch,
        compiler_params=_SC_PARAMS,
    )(wrapped)
    if with_q:
        return fn(v, src3, dst3, dis)
    return fn(v, src3, dst3)


def _tc1(x_pad, W1, W2, degp):
    def body(x_ref, w1_ref, w2_ref, degp_ref, v1_ref, dis_ref):
        deg = jnp.sum(degp_ref[...], axis=0)
        dis = jnp.where(deg > 0, lax.rsqrt(deg), 0.0)
        w12 = jnp.dot(w1_ref[...], w2_ref[...], preferred_element_type=jnp.float32)
        c = jnp.dot(x_ref[...], w12, preferred_element_type=jnp.float32)
        v1_ref[...] = c * dis[:, None]
        dis_ref[...] = dis[None, :]

    return pl.pallas_call(
        body,
        grid=(N_PAD // BLK,),
        in_specs=[
            pl.BlockSpec((BLK, D), lambda i: (i, 0)),
            pl.BlockSpec((D, 64), lambda i: (0, 0)),
            pl.BlockSpec((64, W_AGG), lambda i: (0, 0)),
            pl.BlockSpec((NW, BLK), lambda i: (0, i)),
        ],
        out_specs=[
            pl.BlockSpec((BLK, W_AGG), lambda i: (i, 0)),
            pl.BlockSpec((1, BLK), lambda i: (0, i)),
        ],
        out_shape=[
            jax.ShapeDtypeStruct((N_PAD, W_AGG), jnp.float32),
            jax.ShapeDtypeStruct((1, N_PAD), jnp.float32),
        ],
    )(x_pad, W1, W2, degp)


def _tc2(p, qp, dis):
    def body(p_ref, qp_ref, dis_ref, v2_ref, q_ref):
        a = p_ref[0] + p_ref[1]
        d = dis_ref[0]
        v2_ref[...] = a * (d * d)[:, None]
        q_ref[...] = (d * jnp.sum(qp_ref[...], axis=0))[None, :]

    return pl.pallas_call(
        body,
        grid=(N_PAD // BLK,),
        in_specs=[
            pl.BlockSpec((2, BLK, W_AGG), lambda i: (0, i, 0)),
            pl.BlockSpec((NW, BLK), lambda i: (0, i)),
            pl.BlockSpec((1, BLK), lambda i: (0, i)),
        ],
        out_specs=[
            pl.BlockSpec((BLK, W_AGG), lambda i: (i, 0)),
            pl.BlockSpec((1, BLK), lambda i: (0, i)),
        ],
        out_shape=[
            jax.ShapeDtypeStruct((N_PAD, W_AGG), jnp.float32),
            jax.ShapeDtypeStruct((1, N_PAD), jnp.float32),
        ],
    )(p, qp, dis)


def _tc3(p, dis, q, b1r, W2, b2r):
    def body(p_ref, dis_ref, q_ref, b1_ref, w2_ref, b2_ref, v3_ref):
        d = dis_ref[0][:, None]
        qc = q_ref[0][:, None]
        bw = jnp.dot(b1_ref[...], w2_ref[...], preferred_element_type=jnp.float32)
        h2 = (p_ref[0] + p_ref[1]) * d + qc * bw + b2_ref[...]
        v3_ref[...] = jax.nn.sigmoid(h2) * d

    return pl.pallas_call(
        body,
        grid=(N_PAD // BLK,),
        in_specs=[
            pl.BlockSpec((2, BLK, W_AGG), lambda i: (0, i, 0)),
            pl.BlockSpec((1, BLK), lambda i: (0, i)),
            pl.BlockSpec((1, BLK), lambda i: (0, i)),
            pl.BlockSpec((1, 64), lambda i: (0, 0)),
            pl.BlockSpec((64, W_AGG), lambda i: (0, 0)),
            pl.BlockSpec((1, W_AGG), lambda i: (0, 0)),
        ],
        out_specs=pl.BlockSpec((BLK, W_AGG), lambda i: (i, 0)),
        out_shape=jax.ShapeDtypeStruct((N_PAD, W_AGG), jnp.float32),
    )(p, dis, q, b1r, W2, b2r)


def _tc4(p, dis):
    def body(p_ref, dis_ref, v4_ref):
        d = dis_ref[0]
        v4_ref[...] = (p_ref[0] + p_ref[1]) * (d * d)[:, None]

    return pl.pallas_call(
        body,
        grid=(N_PAD // BLK,),
        in_specs=[
            pl.BlockSpec((2, BLK, W_AGG), lambda i: (0, i, 0)),
            pl.BlockSpec((1, BLK), lambda i: (0, i)),
        ],
        out_specs=pl.BlockSpec((BLK, W_AGG), lambda i: (i, 0)),
        out_shape=jax.ShapeDtypeStruct((N_PAD, W_AGG), jnp.float32),
    )(p, dis)


def _tc5(p, dis, q, W3, W4, b3r, b4r):
    def body(p_ref, dis_ref, q_ref, w3_ref, w4_ref, b3_ref, b4_ref, out_ref):
        d = dis_ref[0][:, None]
        qc = q_ref[0][:, None]
        w34 = jnp.dot(w3_ref[...], w4_ref[...], preferred_element_type=jnp.float32)
        bw = jnp.dot(b3_ref[...], w4_ref[...], preferred_element_type=jnp.float32)
        t = (p_ref[0] + p_ref[1]) * d
        h4 = jnp.dot(t, w34, preferred_element_type=jnp.float32) + qc * bw + b4_ref[...]
        out_ref[...] = jnp.maximum(h4, 0.0)

    return pl.pallas_call(
        body,
        grid=(N_PAD // BLK,),
        in_specs=[
            pl.BlockSpec((2, BLK, W_AGG), lambda i: (0, i, 0)),
            pl.BlockSpec((1, BLK), lambda i: (0, i)),
            pl.BlockSpec((1, BLK), lambda i: (0, i)),
            pl.BlockSpec((W_AGG, 64), lambda i: (0, 0)),
            pl.BlockSpec((64, D), lambda i: (0, 0)),
            pl.BlockSpec((1, 64), lambda i: (0, 0)),
            pl.BlockSpec((1, D), lambda i: (0, 0)),
        ],
        out_specs=pl.BlockSpec((BLK, D), lambda i: (i, 0)),
        out_shape=jax.ShapeDtypeStruct((N_PAD, D), jnp.float32),
    )(p, dis, q, W3, W4, b3r, b4r)


def kernel(x, edge_index, edge_weights, W1, b1, W2, b2, W3, b3, W4, b4):
    del edge_weights
    loop = jnp.arange(N, dtype=jnp.int32)
    pad = jnp.full((E_PAD - E - N,), N, dtype=jnp.int32)
    src3 = jnp.concatenate([edge_index[0], loop, pad]).reshape(NCHUNK, 128)
    dst3 = jnp.concatenate([edge_index[1], loop, pad]).reshape(NCHUNK, 128)
    x_pad = jnp.pad(x, ((0, N_PAD - N), (0, 0)))

    degp = _deg_call(dst3)
    v1, dis = _tc1(x_pad, W1, W2, degp)
    p, qp = _agg_call(v1, src3, dst3, dis=dis.reshape(N_PAD))
    v2, q = _tc2(p, qp, dis)
    p = _agg_call(v2, src3, dst3)
    v3 = _tc3(p, dis, q, b1.reshape(1, 64), W2, b2.reshape(1, W_AGG))
    p = _agg_call(v3, src3, dst3)
    v4 = _tc4(p, dis)
    p = _agg_call(v4, src3, dst3)
    out = _tc5(p, dis, q, W3, W4, b3.reshape(1, 64), b4.reshape(1, D))
    return out[:N]


# speedup vs baseline: 1.3985x; 1.0136x over previous
import functools

import jax
import jax.numpy as jnp
from jax import lax
from jax.experimental import pallas as pl
from jax.experimental.pallas import tpu as pltpu
from jax.experimental.pallas import tpu_sc as plsc

N = 10000
D = 128
E = 320000
W_AGG = 32
NW = 32
CPT_A = 84
CPT_B = 78
CPT_MAX = max(CPT_A, CPT_B)
NCHUNK = 16 * (CPT_A + CPT_B)
E_PAD = NCHUNK * 128
N_PAD = 10240
RPT = N_PAD // 16
BLK = 1024


def _sc_mesh():
    return plsc.VectorSubcoreMesh(core_axis_name="c", subcore_axis_name="s")


_SC_PARAMS = pltpu.CompilerParams(
    use_tc_tiling_on_sc=False, needs_layout_passes=False)


def _stage_chunks(hbm2, vmem, cid, sid):
    base_a = sid * CPT_A
    base_b = 16 * CPT_A + sid * CPT_B

    @pl.when(cid == 0)
    def _():
        pltpu.sync_copy(hbm2.at[pl.ds(base_a, CPT_A)], vmem.at[pl.ds(0, CPT_A)])

    @pl.when(cid == 1)
    def _():
        pltpu.sync_copy(hbm2.at[pl.ds(base_b, CPT_B)], vmem.at[pl.ds(0, CPT_B)])


def _deg_call(dst2):
    @functools.partial(
        pl.kernel,
        out_type=jax.ShapeDtypeStruct((NW, N_PAD), jnp.float32),
        mesh=_sc_mesh(),
        scratch_types=[
            pltpu.VMEM((CPT_MAX, 128), jnp.int32),
            pltpu.VMEM((N_PAD,), jnp.float32),
        ],
        compiler_params=_SC_PARAMS,
    )
    def deg_kernel(dst_hbm, degp_hbm, dstv, degacc):
        cid = lax.axis_index("c")
        sid = lax.axis_index("s")
        wid = sid * 2 + cid
        ncpt = jnp.where(cid == 0, CPT_A, CPT_B)
        _stage_chunks(dst_hbm, dstv, cid, sid)
        zero16 = jnp.zeros((16,), jnp.float32)

        @pl.loop(0, N_PAD // 16)
        def _zero(i):
            degacc[pl.ds(i * 16, 16)] = zero16

        one16 = jnp.ones((16,), jnp.float32)

        @pl.loop(0, ncpt)
        def _chunk(j):
            for t in range(8):
                idx = dstv[j, pl.ds(t * 16, 16)]
                plsc.addupdate_scatter(degacc, [idx], one16)

        pltpu.sync_copy(degacc, degp_hbm.at[wid])

    return deg_kernel(dst2)


def _agg_call(v, src3, dst3, dis=None):
    with_q = dis is not None
    outs = [jax.ShapeDtypeStruct((2, N_PAD, W_AGG), jnp.float32)]
    scratch = [
        pltpu.VMEM((CPT_MAX, 128), jnp.int32),
        pltpu.VMEM((CPT_MAX, 128), jnp.int32),
        [pltpu.VMEM((128, W_AGG), jnp.float32)] * 2,
        [pltpu.SemaphoreType.DMA] * 2,
        pltpu.VMEM_SHARED((N_PAD, W_AGG), jnp.float32),
    ]
    if with_q:
        outs.append(jax.ShapeDtypeStruct((NW, N_PAD), jnp.float32))
        scratch += [
            pltpu.VMEM((N_PAD,), jnp.float32),
            pltpu.VMEM((N_PAD,), jnp.float32),
        ]

    def body(refs):
        if with_q:
            (v_hbm, src_hbm, dst_hbm, dis_hbm, p_hbm, qp_hbm,
             srcv, dstv, rows, gsems, acc, disv, qacc) = refs
        else:
            (v_hbm, src_hbm, dst_hbm, p_hbm,
             srcv, dstv, rows, gsems, acc) = refs
        cid = lax.axis_index("c")
        sid = lax.axis_index("s")
        wid = sid * 2 + cid
        ncpt = jnp.where(cid == 0, CPT_A, CPT_B)
        _stage_chunks(src_hbm, srcv, cid, sid)
        _stage_chunks(dst_hbm, dstv, cid, sid)
        zero16 = jnp.zeros((16,), jnp.float32)

        @pl.loop(0, 128)
        def _zero_rows(i):
            rows[0][i, pl.ds(0, 16)] = zero16
            rows[0][i, pl.ds(16, 16)] = zero16

        for b in range(RPT // 128):
            pltpu.sync_copy(rows[0], acc.at[pl.ds(sid * RPT + b * 128, 128)])

        if with_q:
            pltpu.sync_copy(dis_hbm, disv)

            @pl.loop(0, N_PAD // 16)
            def _zero_q(i):
                qacc[pl.ds(i * 16, 16)] = zero16

        plsc.subcore_barrier()

        def gather(jj, b):
            pltpu.async_copy(v_hbm.at[srcv.at[jj]], rows[b], gsems[b])

        def gather_wait(jj, b):
            pltpu.make_async_copy(v_hbm.at[srcv.at[jj]], rows[b], gsems[b]).wait()

        gather(0, 0)
        gather(1, 1)

        @pl.loop(0, ncpt, step=2)
        def _chunk(j):
            for b in range(2):
                jj = j + b
                gather_wait(jj, b)
                pltpu.sync_copy(rows[b], acc.at[dstv.at[jj]], add=True)

                @pl.when(jj + 2 < ncpt)
                def _():
                    gather(jj + 2, b)

                if with_q:
                    for t in range(8):
                        si = srcv[jj, pl.ds(t * 16, 16)]
                        di = dstv[jj, pl.ds(t * 16, 16)]
                        dv = plsc.load_gather(disv, [si])
                        plsc.addupdate_scatter(qacc, [di], dv)

        plsc.subcore_barrier()
        pltpu.sync_copy(acc.at[pl.ds(sid * RPT, RPT)],
                        p_hbm.at[cid, pl.ds(sid * RPT, RPT)])
        if with_q:
            pltpu.sync_copy(qacc, qp_hbm.at[wid])

    def wrapped(*refs):
        body(refs)

    fn = functools.partial(
        pl.kernel,
        out_type=tuple(outs) if with_q else outs[0],
        mesh=_sc_mesh(),
        scratch_types=scratch,
        compiler_params=_SC_PARAMS,
    )(wrapped)
    if with_q:
        return fn(v, src3, dst3, dis)
    return fn(v, src3, dst3)


def _tc1(x_pad, W1, W2, degp):
    def body(x_ref, w1_ref, w2_ref, degp_ref, v1_ref, dis_ref):
        deg = jnp.sum(degp_ref[...], axis=0)
        dis = jnp.where(deg > 0, lax.rsqrt(deg), 0.0)
        w12 = jnp.dot(w1_ref[...], w2_ref[...], preferred_element_type=jnp.float32)
        c = jnp.dot(x_ref[...], w12, preferred_element_type=jnp.float32)
        v1_ref[...] = c * dis[:, None]
        dis_ref[...] = dis[None, :]

    return pl.pallas_call(
        body,
        grid=(N_PAD // BLK,),
        in_specs=[
            pl.BlockSpec((BLK, D), lambda i: (i, 0)),
            pl.BlockSpec((D, 64), lambda i: (0, 0)),
            pl.BlockSpec((64, W_AGG), lambda i: (0, 0)),
            pl.BlockSpec((NW, BLK), lambda i: (0, i)),
        ],
        out_specs=[
            pl.BlockSpec((BLK, W_AGG), lambda i: (i, 0)),
            pl.BlockSpec((1, BLK), lambda i: (0, i)),
        ],
        out_shape=[
            jax.ShapeDtypeStruct((N_PAD, W_AGG), jnp.float32),
            jax.ShapeDtypeStruct((1, N_PAD), jnp.float32),
        ],
    )(x_pad, W1, W2, degp)


def _tc2(p, qp, dis):
    def body(p_ref, qp_ref, dis_ref, v2_ref, q_ref):
        a = p_ref[0] + p_ref[1]
        d = dis_ref[0]
        v2_ref[...] = a * (d * d)[:, None]
        q_ref[...] = (d * jnp.sum(qp_ref[...], axis=0))[None, :]

    return pl.pallas_call(
        body,
        grid=(N_PAD // BLK,),
        in_specs=[
            pl.BlockSpec((2, BLK, W_AGG), lambda i: (0, i, 0)),
            pl.BlockSpec((NW, BLK), lambda i: (0, i)),
            pl.BlockSpec((1, BLK), lambda i: (0, i)),
        ],
        out_specs=[
            pl.BlockSpec((BLK, W_AGG), lambda i: (i, 0)),
            pl.BlockSpec((1, BLK), lambda i: (0, i)),
        ],
        out_shape=[
            jax.ShapeDtypeStruct((N_PAD, W_AGG), jnp.float32),
            jax.ShapeDtypeStruct((1, N_PAD), jnp.float32),
        ],
    )(p, qp, dis)


def _tc3(p, dis, q, b1r, W2, b2r):
    def body(p_ref, dis_ref, q_ref, b1_ref, w2_ref, b2_ref, v3_ref):
        d = dis_ref[0][:, None]
        qc = q_ref[0][:, None]
        bw = jnp.dot(b1_ref[...], w2_ref[...], preferred_element_type=jnp.float32)
        h2 = (p_ref[0] + p_ref[1]) * d + qc * bw + b2_ref[...]
        v3_ref[...] = jax.nn.sigmoid(h2) * d

    return pl.pallas_call(
        body,
        grid=(N_PAD // BLK,),
        in_specs=[
            pl.BlockSpec((2, BLK, W_AGG), lambda i: (0, i, 0)),
            pl.BlockSpec((1, BLK), lambda i: (0, i)),
            pl.BlockSpec((1, BLK), lambda i: (0, i)),
            pl.BlockSpec((1, 64), lambda i: (0, 0)),
            pl.BlockSpec((64, W_AGG), lambda i: (0, 0)),
            pl.BlockSpec((1, W_AGG), lambda i: (0, 0)),
        ],
        out_specs=pl.BlockSpec((BLK, W_AGG), lambda i: (i, 0)),
        out_shape=jax.ShapeDtypeStruct((N_PAD, W_AGG), jnp.float32),
    )(p, dis, q, b1r, W2, b2r)


def _tc4(p, dis):
    def body(p_ref, dis_ref, v4_ref):
        d = dis_ref[0]
        v4_ref[...] = (p_ref[0] + p_ref[1]) * (d * d)[:, None]

    return pl.pallas_call(
        body,
        grid=(N_PAD // BLK,),
        in_specs=[
            pl.BlockSpec((2, BLK, W_AGG), lambda i: (0, i, 0)),
            pl.BlockSpec((1, BLK), lambda i: (0, i)),
        ],
        out_specs=pl.BlockSpec((BLK, W_AGG), lambda i: (i, 0)),
        out_shape=jax.ShapeDtypeStruct((N_PAD, W_AGG), jnp.float32),
    )(p, dis)


def _tc5(p, dis, q, W3, W4, b3r, b4r):
    def body(p_ref, dis_ref, q_ref, w3_ref, w4_ref, b3_ref, b4_ref, out_ref):
        d = dis_ref[0][:, None]
        qc = q_ref[0][:, None]
        w34 = jnp.dot(w3_ref[...], w4_ref[...], preferred_element_type=jnp.float32)
        bw = jnp.dot(b3_ref[...], w4_ref[...], preferred_element_type=jnp.float32)
        t = (p_ref[0] + p_ref[1]) * d
        h4 = jnp.dot(t, w34, preferred_element_type=jnp.float32) + qc * bw + b4_ref[...]
        out_ref[...] = jnp.maximum(h4, 0.0)

    return pl.pallas_call(
        body,
        grid=(N_PAD // BLK,),
        in_specs=[
            pl.BlockSpec((2, BLK, W_AGG), lambda i: (0, i, 0)),
            pl.BlockSpec((1, BLK), lambda i: (0, i)),
            pl.BlockSpec((1, BLK), lambda i: (0, i)),
            pl.BlockSpec((W_AGG, 64), lambda i: (0, 0)),
            pl.BlockSpec((64, D), lambda i: (0, 0)),
            pl.BlockSpec((1, 64), lambda i: (0, 0)),
            pl.BlockSpec((1, D), lambda i: (0, 0)),
        ],
        out_specs=pl.BlockSpec((BLK, D), lambda i: (i, 0)),
        out_shape=jax.ShapeDtypeStruct((N_PAD, D), jnp.float32),
    )(p, dis, q, W3, W4, b3r, b4r)


def kernel(x, edge_index, edge_weights, W1, b1, W2, b2, W3, b3, W4, b4):
    del edge_weights
    loop = jnp.arange(N, dtype=jnp.int32)
    pad = jnp.full((E_PAD - E - N,), N, dtype=jnp.int32)
    src3 = jnp.concatenate([edge_index[0], loop, pad]).reshape(NCHUNK, 128)
    dst3 = jnp.concatenate([edge_index[1], loop, pad]).reshape(NCHUNK, 128)
    x_pad = jnp.pad(x, ((0, N_PAD - N), (0, 0)))

    degp = _deg_call(dst3)
    v1, dis = _tc1(x_pad, W1, W2, degp)
    p, qp = _agg_call(v1, src3, dst3, dis=dis.reshape(N_PAD))
    v2, q = _tc2(p, qp, dis)
    p = _agg_call(v2, src3, dst3)
    v3 = _tc3(p, dis, q, b1.reshape(1, 64), W2, b2.reshape(1, W_AGG))
    p = _agg_call(v3, src3, dst3)
    v4 = _tc4(p, dis)
    p = _agg_call(v4, src3, dst3)
    out = _tc5(p, dis, q, W3, W4, b3.reshape(1, 64), b4.reshape(1, D))
    return out[:N]


# speedup vs baseline: 1.4106x; 1.0087x over previous
import functools

import jax
import jax.numpy as jnp
from jax import lax
from jax.experimental import pallas as pl
from jax.experimental.pallas import tpu as pltpu
from jax.experimental.pallas import tpu_sc as plsc

N = 10000
D = 128
E = 320000
W_AGG = 32
NW = 32
CPT_A = 88
CPT_B = 74
CPT_MAX = max(CPT_A, CPT_B)
NCHUNK = 16 * (CPT_A + CPT_B)
E_PAD = NCHUNK * 128
N_PAD = 10240
RPT = N_PAD // 16
BLK = 1024


def _sc_mesh():
    return plsc.VectorSubcoreMesh(core_axis_name="c", subcore_axis_name="s")


_SC_PARAMS = pltpu.CompilerParams(
    use_tc_tiling_on_sc=False, needs_layout_passes=False)


def _stage_chunks(hbm2, vmem, cid, sid):
    base_a = sid * CPT_A
    base_b = 16 * CPT_A + sid * CPT_B

    @pl.when(cid == 0)
    def _():
        pltpu.sync_copy(hbm2.at[pl.ds(base_a, CPT_A)], vmem.at[pl.ds(0, CPT_A)])

    @pl.when(cid == 1)
    def _():
        pltpu.sync_copy(hbm2.at[pl.ds(base_b, CPT_B)], vmem.at[pl.ds(0, CPT_B)])


def _deg_call(dst2):
    @functools.partial(
        pl.kernel,
        out_type=jax.ShapeDtypeStruct((NW, N_PAD), jnp.float32),
        mesh=_sc_mesh(),
        scratch_types=[
            pltpu.VMEM((CPT_MAX, 128), jnp.int32),
            pltpu.VMEM((N_PAD,), jnp.float32),
        ],
        compiler_params=_SC_PARAMS,
    )
    def deg_kernel(dst_hbm, degp_hbm, dstv, degacc):
        cid = lax.axis_index("c")
        sid = lax.axis_index("s")
        wid = sid * 2 + cid
        ncpt = jnp.where(cid == 0, CPT_A, CPT_B)
        _stage_chunks(dst_hbm, dstv, cid, sid)
        zero16 = jnp.zeros((16,), jnp.float32)

        @pl.loop(0, N_PAD // 16)
        def _zero(i):
            degacc[pl.ds(i * 16, 16)] = zero16

        one16 = jnp.ones((16,), jnp.float32)

        @pl.loop(0, ncpt)
        def _chunk(j):
            for t in range(8):
                idx = dstv[j, pl.ds(t * 16, 16)]
                plsc.addupdate_scatter(degacc, [idx], one16)

        pltpu.sync_copy(degacc, degp_hbm.at[wid])

    return deg_kernel(dst2)


def _agg_call(v, src3, dst3, dis=None):
    with_q = dis is not None
    outs = [jax.ShapeDtypeStruct((2, N_PAD, W_AGG), jnp.float32)]
    scratch = [
        pltpu.VMEM((CPT_MAX, 128), jnp.int32),
        pltpu.VMEM((CPT_MAX, 128), jnp.int32),
        [pltpu.VMEM((128, W_AGG), jnp.float32)] * 2,
        [pltpu.SemaphoreType.DMA] * 2,
        pltpu.VMEM_SHARED((N_PAD, W_AGG), jnp.float32),
    ]
    if with_q:
        outs.append(jax.ShapeDtypeStruct((NW, N_PAD), jnp.float32))
        scratch += [
            pltpu.VMEM((N_PAD,), jnp.float32),
            pltpu.VMEM((N_PAD,), jnp.float32),
        ]

    def body(refs):
        if with_q:
            (v_hbm, src_hbm, dst_hbm, dis_hbm, p_hbm, qp_hbm,
             srcv, dstv, rows, gsems, acc, disv, qacc) = refs
        else:
            (v_hbm, src_hbm, dst_hbm, p_hbm,
             srcv, dstv, rows, gsems, acc) = refs
        cid = lax.axis_index("c")
        sid = lax.axis_index("s")
        wid = sid * 2 + cid
        ncpt = jnp.where(cid == 0, CPT_A, CPT_B)
        _stage_chunks(src_hbm, srcv, cid, sid)
        _stage_chunks(dst_hbm, dstv, cid, sid)
        zero16 = jnp.zeros((16,), jnp.float32)

        @pl.loop(0, 128)
        def _zero_rows(i):
            rows[0][i, pl.ds(0, 16)] = zero16
            rows[0][i, pl.ds(16, 16)] = zero16

        for b in range(RPT // 128):
            pltpu.sync_copy(rows[0], acc.at[pl.ds(sid * RPT + b * 128, 128)])

        if with_q:
            pltpu.sync_copy(dis_hbm, disv)

            @pl.loop(0, N_PAD // 16)
            def _zero_q(i):
                qacc[pl.ds(i * 16, 16)] = zero16

        plsc.subcore_barrier()

        def gather(jj, b):
            pltpu.async_copy(v_hbm.at[srcv.at[jj]], rows[b], gsems[b])

        def gather_wait(jj, b):
            pltpu.make_async_copy(v_hbm.at[srcv.at[jj]], rows[b], gsems[b]).wait()

        gather(0, 0)
        gather(1, 1)

        @pl.loop(0, ncpt, step=2)
        def _chunk(j):
            for b in range(2):
                jj = j + b
                gather_wait(jj, b)
                pltpu.sync_copy(rows[b], acc.at[dstv.at[jj]], add=True)

                @pl.when(jj + 2 < ncpt)
                def _():
                    gather(jj + 2, b)

                if with_q:
                    for t in range(8):
                        si = srcv[jj, pl.ds(t * 16, 16)]
                        di = dstv[jj, pl.ds(t * 16, 16)]
                        dv = plsc.load_gather(disv, [si])
                        plsc.addupdate_scatter(qacc, [di], dv)

        plsc.subcore_barrier()
        pltpu.sync_copy(acc.at[pl.ds(sid * RPT, RPT)],
                        p_hbm.at[cid, pl.ds(sid * RPT, RPT)])
        if with_q:
            pltpu.sync_copy(qacc, qp_hbm.at[wid])

    def wrapped(*refs):
        body(refs)

    fn = functools.partial(
        pl.kernel,
        out_type=tuple(outs) if with_q else outs[0],
        mesh=_sc_mesh(),
        scratch_types=scratch,
        compiler_params=_SC_PARAMS,
    )(wrapped)
    if with_q:
        return fn(v, src3, dst3, dis)
    return fn(v, src3, dst3)


def _tc1(x_pad, W1, W2, degp):
    def body(x_ref, w1_ref, w2_ref, degp_ref, v1_ref, dis_ref):
        deg = jnp.sum(degp_ref[...], axis=0)
        dis = jnp.where(deg > 0, lax.rsqrt(deg), 0.0)
        w12 = jnp.dot(w1_ref[...], w2_ref[...], preferred_element_type=jnp.float32)
        c = jnp.dot(x_ref[...], w12, preferred_element_type=jnp.float32)
        v1_ref[...] = c * dis[:, None]
        dis_ref[...] = dis[None, :]

    return pl.pallas_call(
        body,
        grid=(N_PAD // BLK,),
        in_specs=[
            pl.BlockSpec((BLK, D), lambda i: (i, 0)),
            pl.BlockSpec((D, 64), lambda i: (0, 0)),
            pl.BlockSpec((64, W_AGG), lambda i: (0, 0)),
            pl.BlockSpec((NW, BLK), lambda i: (0, i)),
        ],
        out_specs=[
            pl.BlockSpec((BLK, W_AGG), lambda i: (i, 0)),
            pl.BlockSpec((1, BLK), lambda i: (0, i)),
        ],
        out_shape=[
            jax.ShapeDtypeStruct((N_PAD, W_AGG), jnp.float32),
            jax.ShapeDtypeStruct((1, N_PAD), jnp.float32),
        ],
    )(x_pad, W1, W2, degp)


def _tc2(p, qp, dis):
    def body(p_ref, qp_ref, dis_ref, v2_ref, q_ref):
        a = p_ref[0] + p_ref[1]
        d = dis_ref[0]
        v2_ref[...] = a * (d * d)[:, None]
        q_ref[...] = (d * jnp.sum(qp_ref[...], axis=0))[None, :]

    return pl.pallas_call(
        body,
        grid=(N_PAD // BLK,),
        in_specs=[
            pl.BlockSpec((2, BLK, W_AGG), lambda i: (0, i, 0)),
            pl.BlockSpec((NW, BLK), lambda i: (0, i)),
            pl.BlockSpec((1, BLK), lambda i: (0, i)),
        ],
        out_specs=[
            pl.BlockSpec((BLK, W_AGG), lambda i: (i, 0)),
            pl.BlockSpec((1, BLK), lambda i: (0, i)),
        ],
        out_shape=[
            jax.ShapeDtypeStruct((N_PAD, W_AGG), jnp.float32),
            jax.ShapeDtypeStruct((1, N_PAD), jnp.float32),
        ],
    )(p, qp, dis)


def _tc3(p, dis, q, b1r, W2, b2r):
    def body(p_ref, dis_ref, q_ref, b1_ref, w2_ref, b2_ref, v3_ref):
        d = dis_ref[0][:, None]
        qc = q_ref[0][:, None]
        bw = jnp.dot(b1_ref[...], w2_ref[...], preferred_element_type=jnp.float32)
        h2 = (p_ref[0] + p_ref[1]) * d + qc * bw + b2_ref[...]
        v3_ref[...] = jax.nn.sigmoid(h2) * d

    return pl.pallas_call(
        body,
        grid=(N_PAD // BLK,),
        in_specs=[
            pl.BlockSpec((2, BLK, W_AGG), lambda i: (0, i, 0)),
            pl.BlockSpec((1, BLK), lambda i: (0, i)),
            pl.BlockSpec((1, BLK), lambda i: (0, i)),
            pl.BlockSpec((1, 64), lambda i: (0, 0)),
            pl.BlockSpec((64, W_AGG), lambda i: (0, 0)),
            pl.BlockSpec((1, W_AGG), lambda i: (0, 0)),
        ],
        out_specs=pl.BlockSpec((BLK, W_AGG), lambda i: (i, 0)),
        out_shape=jax.ShapeDtypeStruct((N_PAD, W_AGG), jnp.float32),
    )(p, dis, q, b1r, W2, b2r)


def _tc4(p, dis):
    def body(p_ref, dis_ref, v4_ref):
        d = dis_ref[0]
        v4_ref[...] = (p_ref[0] + p_ref[1]) * (d * d)[:, None]

    return pl.pallas_call(
        body,
        grid=(N_PAD // BLK,),
        in_specs=[
            pl.BlockSpec((2, BLK, W_AGG), lambda i: (0, i, 0)),
            pl.BlockSpec((1, BLK), lambda i: (0, i)),
        ],
        out_specs=pl.BlockSpec((BLK, W_AGG), lambda i: (i, 0)),
        out_shape=jax.ShapeDtypeStruct((N_PAD, W_AGG), jnp.float32),
    )(p, dis)


def _tc5(p, dis, q, W3, W4, b3r, b4r):
    def body(p_ref, dis_ref, q_ref, w3_ref, w4_ref, b3_ref, b4_ref, out_ref):
        d = dis_ref[0][:, None]
        qc = q_ref[0][:, None]
        w34 = jnp.dot(w3_ref[...], w4_ref[...], preferred_element_type=jnp.float32)
        bw = jnp.dot(b3_ref[...], w4_ref[...], preferred_element_type=jnp.float32)
        t = (p_ref[0] + p_ref[1]) * d
        h4 = jnp.dot(t, w34, preferred_element_type=jnp.float32) + qc * bw + b4_ref[...]
        out_ref[...] = jnp.maximum(h4, 0.0)

    return pl.pallas_call(
        body,
        grid=(N_PAD // BLK,),
        in_specs=[
            pl.BlockSpec((2, BLK, W_AGG), lambda i: (0, i, 0)),
            pl.BlockSpec((1, BLK), lambda i: (0, i)),
            pl.BlockSpec((1, BLK), lambda i: (0, i)),
            pl.BlockSpec((W_AGG, 64), lambda i: (0, 0)),
            pl.BlockSpec((64, D), lambda i: (0, 0)),
            pl.BlockSpec((1, 64), lambda i: (0, 0)),
            pl.BlockSpec((1, D), lambda i: (0, 0)),
        ],
        out_specs=pl.BlockSpec((BLK, D), lambda i: (i, 0)),
        out_shape=jax.ShapeDtypeStruct((N_PAD, D), jnp.float32),
    )(p, dis, q, W3, W4, b3r, b4r)


def kernel(x, edge_index, edge_weights, W1, b1, W2, b2, W3, b3, W4, b4):
    del edge_weights
    loop = jnp.arange(N, dtype=jnp.int32)
    pad = jnp.full((E_PAD - E - N,), N, dtype=jnp.int32)
    src3 = jnp.concatenate([edge_index[0], loop, pad]).reshape(NCHUNK, 128)
    dst3 = jnp.concatenate([edge_index[1], loop, pad]).reshape(NCHUNK, 128)
    x_pad = jnp.pad(x, ((0, N_PAD - N), (0, 0)))

    degp = _deg_call(dst3)
    v1, dis = _tc1(x_pad, W1, W2, degp)
    p, qp = _agg_call(v1, src3, dst3, dis=dis.reshape(N_PAD))
    v2, q = _tc2(p, qp, dis)
    p = _agg_call(v2, src3, dst3)
    v3 = _tc3(p, dis, q, b1.reshape(1, 64), W2, b2.reshape(1, W_AGG))
    p = _agg_call(v3, src3, dst3)
    v4 = _tc4(p, dis)
    p = _agg_call(v4, src3, dst3)
    out = _tc5(p, dis, q, W3, W4, b3.reshape(1, 64), b4.reshape(1, D))
    return out[:N]
